# baseline (speedup 1.0000x reference)
import jax
import jax.numpy as jnp
from jax import lax
from jax.experimental import pallas as pl
from jax.experimental.pallas import tpu as pltpu
from jax.experimental.pallas import tpu_sc as plsc

BATCH = 4096
SEQ = 200
HIDDEN = 64
NUM_WORKERS = 32
BT = 128
LANES = 16
NC = 2


def _sc_body(idx_ref, table_ref, posb_ref, out_ref, idx_v, rows_v, posb_v,
             zb_v, sem_g, sem_o):
    w = lax.axis_index("s") * NC + lax.axis_index("c")
    b0 = w * BT

    iota = lax.iota(jnp.int32, LANES)

    @pl.loop(0, SEQ)
    def _block(s):
        pltpu.sync_copy(idx_ref.at[pl.ds(s * BATCH + b0, BT)], idx_v)
        pltpu.sync_copy(posb_ref.at[pl.ds(s * 8, 8)], posb_v)
        pltpu.async_copy(table_ref.at[idx_v], rows_v, sem_g).wait()

        for ht in range(8):
            for hi in range(8):
                h = ht * 8 + hi
                pvec = posb_v[ht, pl.ds(hi * LANES, LANES)]
                cols16 = jnp.full((LANES,), h, jnp.int32)
                for g in range(8):
                    rows16 = g * LANES + iota
                    v = plsc.load_gather(rows_v, [rows16, cols16])
                    zb_v[ht, 0, hi, pl.ds(g * LANES, LANES)] = v + pvec

        pltpu.async_copy(
            zb_v, out_ref.at[s, :, pl.ds(w, 1)], sem_o).wait()


@jax.jit
def _embed(idx_flat, table, posb):
    mesh = plsc.VectorSubcoreMesh(core_axis_name="c", subcore_axis_name="s")
    f = pl.kernel(
        _sc_body,
        out_type=jax.ShapeDtypeStruct((SEQ, 8, NUM_WORKERS, 8, BT),
                                      jnp.float32),
        mesh=mesh,
        scratch_types=[
            pltpu.VMEM((BT,), jnp.int32),
            pltpu.VMEM((BT, HIDDEN), jnp.float32),
            pltpu.VMEM((8, BT), jnp.float32),
            pltpu.VMEM((8, 1, 8, BT), jnp.float32),
            pltpu.SemaphoreType.DMA,
            pltpu.SemaphoreType.DMA,
        ],
        compiler_params=pltpu.CompilerParams(use_tc_tiling_on_sc=False,
                                             needs_layout_passes=False),
    )
    return f(idx_flat, table, posb)


def kernel(input_ids, word_embeddings, position_embeddings):
    idx_flat = input_ids.T.reshape(-1).astype(jnp.int32)
    posb = jnp.broadcast_to(
        position_embeddings[:SEQ].reshape(SEQ, 8, 8, 1),
        (SEQ, 8, 8, LANES)).reshape(SEQ * 8, 8 * LANES)
    z = _embed(idx_flat, word_embeddings, posb)
    return z.transpose(2, 4, 0, 1, 3).reshape(BATCH, SEQ, HIDDEN)


# speedup vs baseline: 1.2304x; 1.2304x over previous
import jax
import jax.numpy as jnp
from jax import lax
from jax.experimental import pallas as pl
from jax.experimental.pallas import tpu as pltpu
from jax.experimental.pallas import tpu_sc as plsc

BATCH = 4096
SEQ = 200
HIDDEN = 64
NUM_WORKERS = 32
BT = 128
LANES = 16
NC = 2
GRP = 2
NGRP = SEQ // GRP
PSTRIDE = HIDDEN + 1


def _sc_body(idx_ref, table_ref, posb_ref, out_ref,
             idx0, idx1, posb0, posb1, rows0, rows1, rows_p, zb0, zb1,
             sem_g0, sem_g1, sem_o):
    w = lax.axis_index("s") * NC + lax.axis_index("c")

    idxv = (idx0, idx1)
    posbv = (posb0, posb1)
    rows = (rows0, rows1)
    zb = (zb0, zb1)
    sem_g = (sem_g0, sem_g1)

    iota = lax.iota(jnp.int32, LANES)

    def stage(g, buf):
        pltpu.sync_copy(idx_ref.at[w, pl.ds(g * GRP, GRP), :], idxv[buf])
        pltpu.sync_copy(posb_ref.at[pl.ds(g * GRP * 8, GRP * 8)], posbv[buf])
        for k in range(GRP):
            pltpu.async_copy(table_ref.at[idxv[buf].at[k]],
                             rows[buf].at[k], sem_g[buf])

    def out_dma(g, buf):
        return pltpu.async_copy(
            zb[buf], out_ref.at[pl.ds(g * GRP, GRP), :, pl.ds(w, 1), :, :],
            sem_o)

    def process(g, buf):
        for k in range(GRP):
            @pl.loop(0, BT // 4)
            def _pad(b4):
                b = b4 * 4
                for db in range(4):
                    for cg in range(HIDDEN // LANES):
                        v = rows[buf][k, b + db, pl.ds(cg * LANES, LANES)]
                        plsc.store_scatter(
                            rows_p,
                            [(b + db) * PSTRIDE + cg * LANES + iota], v)

            for ht in range(8):
                for hi in range(8):
                    h = ht * 8 + hi
                    pvec = posbv[buf][k * 8 + ht, pl.ds(hi * LANES, LANES)]
                    for bg in range(BT // LANES):
                        ridx = (bg * LANES) * PSTRIDE + h + iota * PSTRIDE
                        v = plsc.load_gather(rows_p, [ridx])
                        zb[buf][k, ht, 0, hi, pl.ds(bg * LANES, LANES)] = (
                            v + pvec)

    stage(0, 0)

    @pl.loop(0, NGRP // 2)
    def _outer(g2):
        for half in range(2):
            buf = half
            g = g2 * 2 + half

            @pl.when(g < NGRP - 1)
            def _prefetch():
                stage(g + 1, 1 - buf)

            for k in range(GRP):
                pltpu.make_async_copy(table_ref.at[idxv[buf].at[k]],
                                      rows[buf].at[k], sem_g[buf]).wait()
            @pl.when(g >= 2)
            def _drain_out():
                out_dma(g - 2, buf).wait()

            process(g, buf)
            out_dma(g, buf)

    out_dma(NGRP - 2, 0).wait()
    out_dma(NGRP - 1, 1).wait()


@jax.jit
def _embed(idx3, table, posb):
    mesh = plsc.VectorSubcoreMesh(core_axis_name="c", subcore_axis_name="s")
    f = pl.kernel(
        _sc_body,
        out_type=jax.ShapeDtypeStruct((SEQ, 8, NUM_WORKERS, 8, BT),
                                      jnp.float32),
        mesh=mesh,
        scratch_types=[
            pltpu.VMEM((GRP, BT), jnp.int32),
            pltpu.VMEM((GRP, BT), jnp.int32),
            pltpu.VMEM((GRP * 8, 8 * LANES), jnp.float32),
            pltpu.VMEM((GRP * 8, 8 * LANES), jnp.float32),
            pltpu.VMEM((GRP, BT, HIDDEN), jnp.float32),
            pltpu.VMEM((GRP, BT, HIDDEN), jnp.float32),
            pltpu.VMEM((BT * PSTRIDE,), jnp.float32),
            pltpu.VMEM((GRP, 8, 1, 8, BT), jnp.float32),
            pltpu.VMEM((GRP, 8, 1, 8, BT), jnp.float32),
            pltpu.SemaphoreType.DMA,
            pltpu.SemaphoreType.DMA,
            pltpu.SemaphoreType.DMA,
        ],
        compiler_params=pltpu.CompilerParams(use_tc_tiling_on_sc=False,
                                             needs_layout_passes=False),
    )
    return f(idx3, table, posb)


def kernel(input_ids, word_embeddings, position_embeddings):
    idx3 = (input_ids.reshape(NUM_WORKERS, BT, SEQ)
            .transpose(0, 2, 1).astype(jnp.int32))
    posb = jnp.broadcast_to(
        position_embeddings[:SEQ].reshape(SEQ, 8, 8, 1),
        (SEQ, 8, 8, LANES)).reshape(SEQ * 8, 8 * LANES)
    z = _embed(idx3, word_embeddings, posb)
    return (z.transpose(2, 4, 0, 1, 3).reshape(BATCH, SEQ, HIDDEN))


# speedup vs baseline: 2.3335x; 1.8966x over previous
import jax
import jax.numpy as jnp
from jax import lax
from jax.experimental import pallas as pl
from jax.experimental.pallas import tpu as pltpu
from jax.experimental.pallas import tpu_sc as plsc

BATCH = 4096
SEQ = 200
HIDDEN = 64
NUM_WORKERS = 32
BT = 128
LANES = 16
NC = 2
ZPAD = BT + 1


def _sc_body(idx_ref, table_ref, posp_ref, out_ref,
             idx0, idx1, pos0, pos1, rows0, rows1, zb0, zb1,
             sem_g0, sem_g1, sem_o):
    w = lax.axis_index("s") * NC + lax.axis_index("c")

    idxv = (idx0, idx1)
    posv = (pos0, pos1)
    rows = (rows0, rows1)
    zb = (zb0, zb1)
    sem_g = (sem_g0, sem_g1)

    iota = lax.iota(jnp.int32, LANES)
    zerov = jnp.zeros((LANES,), jnp.int32)
    htv = [(g16 * LANES + iota) // 8 for g16 in range(4)]
    hiv = [(g16 * LANES + iota) % 8 for g16 in range(4)]

    def stage(s, buf):
        pltpu.sync_copy(idx_ref.at[w, pl.ds(s, 1), :], idxv[buf])
        pltpu.sync_copy(posp_ref.at[pl.ds(s, 1), :], posv[buf])
        pltpu.async_copy(table_ref.at[idxv[buf].at[0]],
                         rows[buf].at[0], sem_g[buf])

    def out_dma(s, buf):
        return pltpu.async_copy(
            zb[buf].at[:, :, :, pl.ds(0, BT)],
            out_ref.at[s, :, pl.ds(w, 1), :, :], sem_o)

    def process(buf):
        @pl.loop(0, BT // LANES)
        def _tile(bg):
            b0 = bg * LANES
            for g16 in range(4):
                pvec = posv[buf][0, pl.ds(g16 * LANES, LANES)]
                vs = [rows[buf][0, b0 + i, pl.ds(g16 * LANES, LANES)]
                      for i in range(LANES)]
                ws = [v + pvec for v in vs]
                for i in range(LANES):
                    bsplat = jnp.full((LANES,), b0 + i, jnp.int32)
                    plsc.store_scatter(zb[buf],
                                       [htv[g16], zerov, hiv[g16], bsplat],
                                       ws[i])

    stage(0, 0)

    @pl.loop(0, SEQ // 2)
    def _outer(s2):
        for half in range(2):
            buf = half
            s = s2 * 2 + half

            @pl.when(s < SEQ - 1)
            def _prefetch():
                stage(s + 1, 1 - buf)

            pltpu.make_async_copy(table_ref.at[idxv[buf].at[0]],
                                  rows[buf].at[0], sem_g[buf]).wait()

            @pl.when(s >= 2)
            def _drain_out():
                out_dma(s - 2, buf).wait()

            process(buf)
            out_dma(s, buf)

    out_dma(SEQ - 2, 0).wait()
    out_dma(SEQ - 1, 1).wait()


@jax.jit
def _embed(idx3, table, posp):
    mesh = plsc.VectorSubcoreMesh(core_axis_name="c", subcore_axis_name="s")
    f = pl.kernel(
        _sc_body,
        out_type=jax.ShapeDtypeStruct((SEQ, 8, NUM_WORKERS, 8, BT),
                                      jnp.float32),
        mesh=mesh,
        scratch_types=[
            pltpu.VMEM((1, BT), jnp.int32),
            pltpu.VMEM((1, BT), jnp.int32),
            pltpu.VMEM((1, BT), jnp.float32),
            pltpu.VMEM((1, BT), jnp.float32),
            pltpu.VMEM((1, BT, HIDDEN), jnp.float32),
            pltpu.VMEM((1, BT, HIDDEN), jnp.float32),
            pltpu.VMEM((8, 1, 8, ZPAD), jnp.float32),
            pltpu.VMEM((8, 1, 8, ZPAD), jnp.float32),
            pltpu.SemaphoreType.DMA,
            pltpu.SemaphoreType.DMA,
            pltpu.SemaphoreType.DMA,
        ],
        compiler_params=pltpu.CompilerParams(use_tc_tiling_on_sc=False,
                                             needs_layout_passes=False),
    )
    return f(idx3, table, posp)


def kernel(input_ids, word_embeddings, position_embeddings):
    idx3 = (input_ids.reshape(NUM_WORKERS, BT, SEQ)
            .transpose(0, 2, 1).astype(jnp.int32))
    posp = jnp.pad(position_embeddings[:SEQ],
                   ((0, 0), (0, BT - HIDDEN)))
    z = _embed(idx3, word_embeddings, posp)
    return (z.transpose(2, 4, 0, 1, 3).reshape(BATCH, SEQ, HIDDEN))


# speedup vs baseline: 2.4466x; 1.0485x over previous
import jax
import jax.numpy as jnp
from jax import lax
from jax.experimental import pallas as pl
from jax.experimental.pallas import tpu as pltpu
from jax.experimental.pallas import tpu_sc as plsc

BATCH = 4096
SEQ = 200
HIDDEN = 64
NUM_WORKERS = 32
BT = 128
LANES = 16
NC = 2
ZPAD = BT + 1


def _sc_body(idx_ref, table_ref, posp_ref, out_ref,
             idx0, idx1, pos0, pos1, rows0, rows1, zb0, zb1,
             sem_g0, sem_g1, sem_o):
    w = lax.axis_index("s") * NC + lax.axis_index("c")

    idxv = (idx0, idx1)
    posv = (pos0, pos1)
    rows = (rows0, rows1)
    zb = (zb0, zb1)
    sem_g = (sem_g0, sem_g1)

    iota = lax.iota(jnp.int32, LANES)
    zerov = jnp.zeros((LANES,), jnp.int32)
    htv = [(g16 * LANES + iota) // 8 for g16 in range(4)]
    hiv = [(g16 * LANES + iota) % 8 for g16 in range(4)]

    def stage(s, buf):
        pltpu.sync_copy(idx_ref.at[w, pl.ds(s, 1), :], idxv[buf])
        pltpu.sync_copy(posp_ref.at[pl.ds(s, 1), :], posv[buf])
        pltpu.async_copy(table_ref.at[idxv[buf].at[0]],
                         rows[buf].at[0], sem_g[buf])

    def out_dma(s, buf):
        return pltpu.async_copy(
            zb[buf].at[:, :, :, pl.ds(0, BT)],
            out_ref.at[s, :, pl.ds(w, 1), :, :], sem_o)

    def out_wait(s, buf):
        pltpu.make_async_copy(
            zb[buf].at[:, :, :, pl.ds(0, BT)],
            out_ref.at[s, :, pl.ds(w, 1), :, :], sem_o).wait()

    def process(buf):
        @pl.loop(0, BT // LANES)
        def _tile(bg):
            b0 = bg * LANES
            for g16 in range(4):
                pvec = posv[buf][0, pl.ds(g16 * LANES, LANES)]
                vs = [rows[buf][0, b0 + i, pl.ds(g16 * LANES, LANES)]
                      for i in range(LANES)]
                ws = [v + pvec for v in vs]
                for i in range(LANES):
                    bsplat = jnp.full((LANES,), b0 + i, jnp.int32)
                    plsc.store_scatter(zb[buf],
                                       [htv[g16], zerov, hiv[g16], bsplat],
                                       ws[i])

    stage(0, 0)

    @pl.loop(0, SEQ // 2)
    def _outer(s2):
        for half in range(2):
            buf = half
            s = s2 * 2 + half

            @pl.when(s < SEQ - 1)
            def _prefetch():
                stage(s + 1, 1 - buf)

            pltpu.make_async_copy(table_ref.at[idxv[buf].at[0]],
                                  rows[buf].at[0], sem_g[buf]).wait()

            @pl.when(s >= 2)
            def _drain_out():
                out_wait(s - 2, buf)

            process(buf)
            out_dma(s, buf)

    out_wait(SEQ - 2, 0)
    out_wait(SEQ - 1, 1)


@jax.jit
def _embed(idx3, table, posp):
    mesh = plsc.VectorSubcoreMesh(core_axis_name="c", subcore_axis_name="s")
    f = pl.kernel(
        _sc_body,
        out_type=jax.ShapeDtypeStruct((SEQ, 8, NUM_WORKERS, 8, BT),
                                      jnp.float32),
        mesh=mesh,
        scratch_types=[
            pltpu.VMEM((1, BT), jnp.int32),
            pltpu.VMEM((1, BT), jnp.int32),
            pltpu.VMEM((1, BT), jnp.float32),
            pltpu.VMEM((1, BT), jnp.float32),
            pltpu.VMEM((1, BT, HIDDEN), jnp.float32),
            pltpu.VMEM((1, BT, HIDDEN), jnp.float32),
            pltpu.VMEM((8, 1, 8, ZPAD), jnp.float32),
            pltpu.VMEM((8, 1, 8, ZPAD), jnp.float32),
            pltpu.SemaphoreType.DMA,
            pltpu.SemaphoreType.DMA,
            pltpu.SemaphoreType.DMA,
        ],
        compiler_params=pltpu.CompilerParams(use_tc_tiling_on_sc=False,
                                             needs_layout_passes=False),
    )
    return f(idx3, table, posp)


def kernel(input_ids, word_embeddings, position_embeddings):
    idx3 = (input_ids.reshape(NUM_WORKERS, BT, SEQ)
            .transpose(0, 2, 1).astype(jnp.int32))
    posp = jnp.pad(position_embeddings[:SEQ],
                   ((0, 0), (0, BT - HIDDEN)))
    z = _embed(idx3, word_embeddings, posp)
    return (z.transpose(2, 4, 0, 1, 3).reshape(BATCH, SEQ, HIDDEN))


# speedup vs baseline: 2.5825x; 1.0555x over previous
import jax
import jax.numpy as jnp
from jax import lax
from jax.experimental import pallas as pl
from jax.experimental.pallas import tpu as pltpu
from jax.experimental.pallas import tpu_sc as plsc

BATCH = 4096
SEQ = 200
HIDDEN = 64
NUM_WORKERS = 32
BT = 128
LANES = 16
NC = 2
ZPAD = BT + 1


def _sc_body(idx_ref, table_ref, posp_ref, out_ref,
             idx0, idx1, pos0, pos1, rows0, rows1, zb0, zb1,
             sem_g0, sem_g1, sem_o):
    w = lax.axis_index("s") * NC + lax.axis_index("c")

    idxv = (idx0, idx1)
    posv = (pos0, pos1)
    rows = (rows0, rows1)
    zb = (zb0, zb1)
    sem_g = (sem_g0, sem_g1)

    iota = lax.iota(jnp.int32, LANES)
    zerov = jnp.zeros((LANES,), jnp.int32)
    htv = [(g16 * LANES + iota) // 8 for g16 in range(4)]
    hiv = [(g16 * LANES + iota) % 8 for g16 in range(4)]

    def stage(s, buf):
        pltpu.sync_copy(idx_ref.at[w, pl.ds(s, 1), :], idxv[buf])
        pltpu.sync_copy(posp_ref.at[pl.ds(s, 1), :], posv[buf])
        pltpu.async_copy(table_ref.at[idxv[buf].at[0]],
                         rows[buf].at[0], sem_g[buf])

    def out_dma(s, buf):
        return pltpu.async_copy(
            zb[buf].at[:, :, :, pl.ds(0, BT)],
            out_ref.at[s, :, pl.ds(w, 1), :, :], sem_o)

    def out_wait(s, buf):
        pltpu.make_async_copy(
            zb[buf].at[:, :, :, pl.ds(0, BT)],
            out_ref.at[s, :, pl.ds(w, 1), :, :], sem_o).wait()

    def process(buf):
        @pl.loop(0, BT // LANES)
        def _tile(bg):
            b0 = bg * LANES
            for g16 in range(4):
                pvec = posv[buf][0, pl.ds(g16 * LANES, LANES)]
                vs = [rows[buf][0, b0 + i, pl.ds(g16 * LANES, LANES)]
                      for i in range(LANES)]
                ws = [v + pvec for v in vs]
                for i in range(LANES):
                    bsplat = jnp.full((LANES,), b0 + i, jnp.int32)
                    plsc.store_scatter(zb[buf],
                                       [htv[g16], zerov, hiv[g16], bsplat],
                                       ws[i])

    stage(0, 0)

    @pl.loop(0, SEQ // 2)
    def _outer(s2):
        for half in range(2):
            buf = half
            s = s2 * 2 + half

            @pl.when(s < SEQ - 1)
            def _prefetch():
                stage(s + 1, 1 - buf)

            pltpu.make_async_copy(table_ref.at[idxv[buf].at[0]],
                                  rows[buf].at[0], sem_g[buf]).wait()

            @pl.when(s >= 2)
            def _drain_out():
                out_wait(s - 2, buf)

            process(buf)
            out_dma(s, buf)

    out_wait(SEQ - 2, 0)
    out_wait(SEQ - 1, 1)


@jax.jit
def _embed(idx3, table, posp):
    mesh = plsc.VectorSubcoreMesh(core_axis_name="c", subcore_axis_name="s")
    f = pl.kernel(
        _sc_body,
        out_type=jax.ShapeDtypeStruct((SEQ, 8, NUM_WORKERS, 8, BT),
                                      jnp.float32),
        mesh=mesh,
        scratch_types=[
            pltpu.VMEM((1, BT), jnp.int32),
            pltpu.VMEM((1, BT), jnp.int32),
            pltpu.VMEM((1, BT), jnp.float32),
            pltpu.VMEM((1, BT), jnp.float32),
            pltpu.VMEM((1, BT, BT), jnp.float32),
            pltpu.VMEM((1, BT, BT), jnp.float32),
            pltpu.VMEM((8, 1, 8, ZPAD), jnp.float32),
            pltpu.VMEM((8, 1, 8, ZPAD), jnp.float32),
            pltpu.SemaphoreType.DMA,
            pltpu.SemaphoreType.DMA,
            pltpu.SemaphoreType.DMA,
        ],
        compiler_params=pltpu.CompilerParams(use_tc_tiling_on_sc=False,
                                             needs_layout_passes=False),
    )
    return f(idx3, table, posp)


def kernel(input_ids, word_embeddings, position_embeddings):
    idx3 = (input_ids.reshape(NUM_WORKERS, BT, SEQ)
            .transpose(0, 2, 1).astype(jnp.int32))
    posp = jnp.pad(position_embeddings[:SEQ],
                   ((0, 0), (0, BT - HIDDEN)))
    wpad = jnp.pad(word_embeddings, ((0, 0), (0, BT - HIDDEN)))
    z = _embed(idx3, wpad, posp)
    return (z.transpose(2, 4, 0, 1, 3).reshape(BATCH, SEQ, HIDDEN))
